# Initial kernel scaffold; baseline (speedup 1.0000x reference)
import jax
import jax.numpy as jnp
from jax.experimental import pallas as pl


def kernel(input, table):
    raise NotImplementedError("write your pallas kernel here")


# speedup vs baseline: 2.3774x; 2.3774x over previous
import functools

import jax
import jax.numpy as jnp
from jax import lax
from jax.experimental import pallas as pl
from jax.experimental.pallas import tpu as pltpu
from jax.experimental.pallas import tpu_sc as plsc

_NC = 2
_NS = 16
_NW = _NC * _NS


def _gather_call(total, V, D, C):
    n_chunks_w = total // (_NW * C)
    b_per_w = total // _NW
    mesh = plsc.VectorSubcoreMesh(core_axis_name="c", subcore_axis_name="s")

    @functools.partial(
        pl.kernel,
        out_type=jax.ShapeDtypeStruct((total, D), jnp.float32),
        mesh=mesh,
        scratch_types=[
            pltpu.VMEM((C,), jnp.int32),
            pltpu.VMEM((C, D), jnp.float32),
            pltpu.SemaphoreType.DMA,
        ],
        compiler_params=pltpu.CompilerParams(use_tc_tiling_on_sc=False),
    )
    def body(idx_hbm, table_hbm, out_hbm, idx_v, rows_v, sem):
        wid = lax.axis_index("s") * _NC + lax.axis_index("c")
        base = wid * b_per_w

        @pl.loop(0, n_chunks_w)
        def _(i):
            off = base + i * C
            pltpu.sync_copy(idx_hbm.at[pl.ds(off, C)], idx_v)
            pltpu.async_copy(table_hbm.at[idx_v], rows_v, sem).wait()
            pltpu.sync_copy(rows_v, out_hbm.at[pl.ds(off, C)])

    return body


def kernel(input, table):
    B, H = input.shape
    V, D = table.shape
    total = B * H
    idx_flat = input.reshape(total).astype(jnp.int32)
    C = 1600
    out = _gather_call(total, V, D, C)(idx_flat, table)
    return out.reshape(B, H * D)


# speedup vs baseline: 2.4329x; 1.0234x over previous
import functools

import jax
import jax.numpy as jnp
from jax import lax
from jax.experimental import pallas as pl
from jax.experimental.pallas import tpu as pltpu
from jax.experimental.pallas import tpu_sc as plsc

_NC = 2
_NS = 16
_NW = _NC * _NS


def _gather_call(total, V, D, C):
    n_chunks_w = total // (_NW * C)
    b_per_w = total // _NW
    mesh = plsc.VectorSubcoreMesh(core_axis_name="c", subcore_axis_name="s")

    n = n_chunks_w

    @functools.partial(
        pl.kernel,
        out_type=jax.ShapeDtypeStruct((total, D), jnp.float32),
        mesh=mesh,
        scratch_types=[
            pltpu.VMEM((b_per_w,), jnp.int32),
            pltpu.VMEM((2, C, D), jnp.float32),
            pltpu.SemaphoreType.DMA,
            pltpu.SemaphoreType.DMA,
            pltpu.SemaphoreType.DMA,
            pltpu.SemaphoreType.DMA,
        ],
        compiler_params=pltpu.CompilerParams(use_tc_tiling_on_sc=False),
    )
    def body(idx_hbm, table_hbm, out_hbm, idx_v, rows_v, g0, g1, w0, w1):
        wid = lax.axis_index("s") * _NC + lax.axis_index("c")
        base = wid * b_per_w
        gsem, wsem = (g0, g1), (w0, w1)
        pltpu.sync_copy(idx_hbm.at[pl.ds(base, b_per_w)], idx_v)

        def start_g(j):
            b = j % 2
            return pltpu.async_copy(
                table_hbm.at[idx_v.at[pl.ds(j * C, C)]], rows_v.at[b], gsem[b])

        def start_w(j):
            b = j % 2
            return pltpu.async_copy(
                rows_v.at[b], out_hbm.at[pl.ds(base + j * C, C)], wsem[b])

        g = [None] * n
        w = [None] * n
        g[0] = start_g(0)
        for j in range(1, n):
            if j >= 2:
                w[j - 2].wait()
            g[j] = start_g(j)
            g[j - 1].wait()
            w[j - 1] = start_w(j - 1)
        g[n - 1].wait()
        w[n - 1] = start_w(n - 1)
        w[n - 2].wait()
        w[n - 1].wait()

    return body


def kernel(input, table):
    B, H = input.shape
    V, D = table.shape
    total = B * H
    idx_flat = input.reshape(total).astype(jnp.int32)
    C = 1600
    out = _gather_call(total, V, D, C)(idx_flat, table)
    return out.reshape(B, H * D)


# speedup vs baseline: 2.7121x; 1.1148x over previous
import functools

import jax
import jax.numpy as jnp
from jax import lax
from jax.experimental import pallas as pl
from jax.experimental.pallas import tpu as pltpu
from jax.experimental.pallas import tpu_sc as plsc

_NC = 2
_NS = 16
_NW = _NC * _NS


def _detile(table_t, rows_per_block):
    D, V = table_t.shape
    nblk = (V + rows_per_block - 1) // rows_per_block
    pack = 128 // D
    prows = rows_per_block // pack

    def body(t_ref, o_ref):
        xt = t_ref[...].T
        x = xt.reshape(prows, pack, D)
        o_ref[...] = jnp.concatenate([x[:, q, :] for q in range(pack)], axis=1)

    return pl.pallas_call(
        body,
        grid=(nblk,),
        in_specs=[pl.BlockSpec((D, rows_per_block), lambda i: (0, i))],
        out_specs=pl.BlockSpec((prows, 128), lambda i: (i, 0)),
        out_shape=jax.ShapeDtypeStruct((V * D // 128, 128), jnp.float32),
    )(table_t)


def _gather_call(total, V, D, C):
    n = total // (_NW * C)
    b_per_w = total // _NW
    mesh = plsc.VectorSubcoreMesh(core_axis_name="c", subcore_axis_name="s")

    @functools.partial(
        pl.kernel,
        out_type=jax.ShapeDtypeStruct((total, D), jnp.float32),
        mesh=mesh,
        scratch_types=[
            pltpu.VMEM((b_per_w,), jnp.int32),
            pltpu.VMEM((2, C, D), jnp.float32),
            pltpu.SemaphoreType.DMA,
            pltpu.SemaphoreType.DMA,
            pltpu.SemaphoreType.DMA,
            pltpu.SemaphoreType.DMA,
        ],
        compiler_params=pltpu.CompilerParams(use_tc_tiling_on_sc=False),
    )
    def body(idx_hbm, table_hbm, out_hbm, idx_v, rows_v, g0, g1, w0, w1):
        wid = lax.axis_index("s") * _NC + lax.axis_index("c")
        base = wid * b_per_w
        gsem = (g0, g1)
        wsem = (w0, w1)
        pltpu.sync_copy(idx_hbm.at[pl.ds(base, b_per_w)], idx_v)

        def start_g(j):
            b = j % 2
            return pltpu.async_copy(
                table_hbm.at[idx_v.at[pl.ds(j * C, C)]], rows_v.at[b], gsem[b])

        def start_w(j):
            b = j % 2
            return pltpu.async_copy(
                rows_v.at[b], out_hbm.at[pl.ds(base + j * C, C)], wsem[b])

        g = [None] * n
        w = [None] * n
        g[0] = start_g(0)
        for j in range(1, n):
            if j >= 2:
                w[j - 2].wait()
            g[j] = start_g(j)
            g[j - 1].wait()
            w[j - 1] = start_w(j - 1)
        g[n - 1].wait()
        w[n - 1] = start_w(n - 1)
        w[n - 2].wait()
        w[n - 1].wait()

    return body


def kernel(input, table):
    B, H = input.shape
    V, D = table.shape
    total = B * H
    idx_flat = input.reshape(total).astype(jnp.int32)
    tlin = _detile(table.T, 4096).reshape(V, D)
    C = 1600
    out = _gather_call(total, V, D, C)(idx_flat, tlin)
    return out.reshape(B, H * D)
